# baseline (speedup 1.0000x reference)
import functools

import jax
import jax.numpy as jnp
from jax import lax
from jax.experimental import pallas as pl
from jax.experimental.pallas import tpu as pltpu
from jax.experimental.pallas import tpu_sc as plsc

N, K, D = 4096, 8192, 256
BN = 256
KC = 2048


CK = 512
NCHUNK = K // CK


def _argmin_body(x_ref, cb_ref, idx_ref, csq_ref, d2_ref, io_ref):
    i = pl.program_id(0)

    @pl.when(i == 0)
    def _():
        sq = cb_ref[...] * cb_ref[...]
        ones = jnp.ones((1, D), jnp.float32)
        csq_ref[...] = lax.dot_general(
            ones, sq, (((1,), (1,)), ((), ())),
            preferred_element_type=jnp.float32)
        io_ref[...] = lax.broadcasted_iota(
            jnp.int32, (BN, K), 1).astype(jnp.float32)

    x = x_ref[...]
    x_sq = jnp.sum(x * x, axis=1, keepdims=True)
    x2 = x + x

    def pass1(c, m):
        cb_c = cb_ref[pl.ds(c * CK, CK), :]
        cross2 = lax.dot_general(
            x2, cb_c, (((1,), (1,)), ((), ())),
            preferred_element_type=jnp.float32)
        raw = (x_sq + csq_ref[0:1, pl.ds(c * CK, CK)]) - cross2
        d2_ref[:, pl.ds(c * CK, CK)] = raw
        return jnp.minimum(m, jnp.min(raw, axis=1, keepdims=True))

    m = lax.fori_loop(0, NCHUNK, pass1,
                      jnp.full((BN, 1), jnp.inf, jnp.float32))
    m = jnp.maximum(m, 1e-12)

    s = jnp.sqrt(m)
    mbits = lax.bitcast_convert_type(m, jnp.int32)
    thr = m
    for j in (1, 2, 3, 4):
        cj = lax.bitcast_convert_type(mbits + j, jnp.float32)
        thr = jnp.where(jnp.sqrt(cj) == s, cj, thr)

    def pass2(c, best):
        raw = d2_ref[:, pl.ds(c * CK, CK)]
        io = io_ref[:, pl.ds(c * CK, CK)]
        cand = jnp.where(raw <= thr, io, jnp.float32(K))
        return jnp.minimum(best, jnp.min(cand, axis=1, keepdims=True))

    bestf = lax.fori_loop(0, NCHUNK, pass2,
                          jnp.full((BN, 1), jnp.float32(K)))
    idx_ref[...] = bestf.astype(jnp.int32)


def _tc_argmin(x, codebook, interpret=False):
    return pl.pallas_call(
        _argmin_body,
        grid=(N // BN,),
        in_specs=[
            pl.BlockSpec((BN, D), lambda i: (i, 0)),
            pl.BlockSpec((K, D), lambda i: (0, 0)),
        ],
        out_specs=pl.BlockSpec((BN, 1), lambda i: (i, 0)),
        out_shape=jax.ShapeDtypeStruct((N, 1), jnp.int32),
        scratch_shapes=[pltpu.VMEM((1, K), jnp.float32),
                        pltpu.VMEM((BN, K), jnp.float32),
                        pltpu.VMEM((BN, K), jnp.float32)],
        compiler_params=pltpu.CompilerParams(
            dimension_semantics=("arbitrary",)),
        interpret=interpret,
    )(x, codebook)


@functools.lru_cache(maxsize=None)
def _make_sc_gather():
    info = plsc.get_sparse_core_info()
    nw = info.num_cores * info.num_subcores
    bpw = N // nw
    mesh = plsc.VectorSubcoreMesh(core_axis_name="c", subcore_axis_name="s")

    @functools.partial(
        pl.kernel,
        mesh=mesh,
        out_type=jax.ShapeDtypeStruct((N, D), jnp.float32),
        scratch_types=[
            pltpu.VMEM((bpw,), jnp.int32),
            pltpu.VMEM((bpw, D), jnp.float32),
            pltpu.SemaphoreType.DMA,
        ],
    )
    def _sc_gather(table_hbm, idx_hbm, out_hbm, idx_v, rows_v, sem):
        wid = lax.axis_index("s") * info.num_cores + lax.axis_index("c")
        base = wid * bpw
        pltpu.sync_copy(idx_hbm.at[pl.ds(base, bpw)], idx_v)
        pltpu.async_copy(table_hbm.at[idx_v], rows_v, sem).wait()
        pltpu.sync_copy(rows_v, out_hbm.at[pl.ds(base, bpw)])

    return _sc_gather


def kernel(x, codebook, embedding_table):
    idx = _tc_argmin(x, codebook).reshape(N)
    return _make_sc_gather()(embedding_table, idx)


# speedup vs baseline: 2.3299x; 2.3299x over previous
import functools

import jax
import jax.numpy as jnp
from jax import lax
from jax.experimental import pallas as pl
from jax.experimental.pallas import tpu as pltpu
from jax.experimental.pallas import tpu_sc as plsc

N, K, D = 4096, 8192, 256
BN = 256
KC = 2048


CK = 512
NCHUNK = K // CK


def _argmin_body(x_ref, cb_ref, idx_ref, csq_ref, io_ref):
    i = pl.program_id(0)

    @pl.when(i == 0)
    def _():
        sq = cb_ref[...] * cb_ref[...]
        ones = jnp.ones((1, D), jnp.float32)
        csq_ref[...] = lax.dot_general(
            ones, sq, (((1,), (1,)), ((), ())),
            preferred_element_type=jnp.float32)
        io_ref[...] = lax.broadcasted_iota(
            jnp.int32, (BN, K), 1).astype(jnp.float32)

    x = x_ref[...]
    x_sq = jnp.sum(x * x, axis=1, keepdims=True)
    x2 = x + x

    cross2 = lax.dot_general(
        x2, cb_ref[...], (((1,), (1,)), ((), ())),
        preferred_element_type=jnp.float32)
    raw = (x_sq + csq_ref[...]) - cross2
    m = jnp.maximum(jnp.min(raw, axis=1, keepdims=True), 1e-12)

    s = jnp.sqrt(m)
    mbits = lax.bitcast_convert_type(m, jnp.int32)
    thr = m
    for j in (1, 2, 3, 4):
        cj = lax.bitcast_convert_type(mbits + j, jnp.float32)
        thr = jnp.where(jnp.sqrt(cj) == s, cj, thr)

    cand = jnp.where(raw <= thr, io_ref[...], jnp.float32(K))
    bestf = jnp.min(cand, axis=1, keepdims=True)
    idx_ref[...] = bestf.astype(jnp.int32)


def _tc_argmin(x, codebook, interpret=False):
    return pl.pallas_call(
        _argmin_body,
        grid=(N // BN,),
        in_specs=[
            pl.BlockSpec((BN, D), lambda i: (i, 0)),
            pl.BlockSpec((K, D), lambda i: (0, 0)),
        ],
        out_specs=pl.BlockSpec((BN, 1), lambda i: (i, 0)),
        out_shape=jax.ShapeDtypeStruct((N, 1), jnp.int32),
        scratch_shapes=[pltpu.VMEM((1, K), jnp.float32),
                        pltpu.VMEM((BN, K), jnp.float32)],
        compiler_params=pltpu.CompilerParams(
            dimension_semantics=("arbitrary",)),
        interpret=interpret,
    )(x, codebook)


@functools.lru_cache(maxsize=None)
def _make_sc_gather():
    info = plsc.get_sparse_core_info()
    nw = info.num_cores * info.num_subcores
    bpw = N // nw
    mesh = plsc.VectorSubcoreMesh(core_axis_name="c", subcore_axis_name="s")

    @functools.partial(
        pl.kernel,
        mesh=mesh,
        out_type=jax.ShapeDtypeStruct((N, D), jnp.float32),
        scratch_types=[
            pltpu.VMEM((bpw,), jnp.int32),
            pltpu.VMEM((bpw, D), jnp.float32),
            pltpu.SemaphoreType.DMA,
        ],
    )
    def _sc_gather(table_hbm, idx_hbm, out_hbm, idx_v, rows_v, sem):
        wid = lax.axis_index("s") * info.num_cores + lax.axis_index("c")
        base = wid * bpw
        pltpu.sync_copy(idx_hbm.at[pl.ds(base, bpw)], idx_v)
        pltpu.async_copy(table_hbm.at[idx_v], rows_v, sem).wait()
        pltpu.sync_copy(rows_v, out_hbm.at[pl.ds(base, bpw)])

    return _sc_gather


def kernel(x, codebook, embedding_table):
    idx = _tc_argmin(x, codebook).reshape(N)
    return _make_sc_gather()(embedding_table, idx)
